# baseline (speedup 1.0000x reference)
import jax
import jax.numpy as jnp
from jax import lax
from jax.experimental import pallas as pl
from jax.experimental.pallas import tpu as pltpu
from jax.experimental.pallas import tpu_sc as plsc

N = 50000
KNB = 8
NC = 2
NS = 16
NW = NC * NS
NPLANE = 6
NSUB = 5
SUB = N // NSUB
CH = 2000
NCHUNK = SUB // CH
CH64 = (CH // 64) * 64


def _diff_body(ci, cp, fi, fp, o_ref):
    for c in range(3):
        o_ref[pl.ds(c * N, N)] = ci[c, :] - cp[c, :]
        o_ref[pl.ds((3 + c) * N, N)] = fi[c, :] - fp[c, :]


def _fire_idx_chunk(idxF, ibuf, g, nstart, sem):
    return [
        pltpu.async_copy(
            idxF.at[pl.ds((k * 2 + g) * N + nstart, CH)],
            ibuf.at[pl.ds(slot * CH, CH)], sem)
        for slot, k in enumerate(list(range(KNB)) + [KNB + 1])
    ]


def _sc_body(d_flat, idxF, out_hbm, table, ibuf0, ibuf1, outv, semt, semi):
    ibufs = (ibuf0, ibuf1)
    wid = lax.axis_index("c") * NS + lax.axis_index("s")
    outv[...] = jnp.zeros((16,), jnp.float32)

    @pl.when(wid < NPLANE * NSUB)
    def _():
        p = wid % NPLANE
        sub = wid // NPLANE
        g = p // 3
        nbase = sub * SUB

        tcopy = pltpu.async_copy(d_flat.at[pl.ds(p * N, N)], table, semt)
        pend = _fire_idx_chunk(idxF, ibufs[0], g, nbase, semi)
        tcopy.wait()
        lossvec = jnp.zeros((16,), jnp.float32)
        for j in range(NCHUNK):
            nxt = None
            if j + 1 < NCHUNK:
                nxt = _fire_idx_chunk(
                    idxF, ibufs[(j + 1) % 2], g, nbase + (j + 1) * CH, semi)
            for h in pend:
                h.wait()
            ibuf = ibufs[j % 2]

            def sub16(o, lv, ibuf=ibuf, j=j):
                deg = ibuf[pl.ds(KNB * CH + o, 16)]
                inv = 1.0 / deg.astype(jnp.float32)
                acc = jnp.zeros((16,), jnp.float32)
                for k in range(KNB):
                    nb = ibuf[pl.ds(k * CH + o, 16)]
                    acc = acc + plsc.load_gather(table, [nb])
                own = table[pl.ds(nbase + j * CH + o, 16)]
                r = own - acc * inv
                return lv + r * r

            def body64(o, lv, sub16=sub16):
                for q in range(4):
                    lv = sub16(o + 16 * q, lv)
                return lv

            lossvec = plsc.parallel_loop(0, CH64, 64, carry=lossvec)(body64)
            for o in range(CH64, CH, 16):
                lossvec = sub16(o, lossvec)
            pend = nxt
        outv[...] = lossvec

    pltpu.sync_copy(outv, out_hbm.at[pl.ds(wid * 16, 16)])


def kernel(coarse_input, coarse_pred, fine_input, fine_pred, laplace_idx_list):
    d_flat = pl.pallas_call(
        _diff_body,
        out_shape=jax.ShapeDtypeStruct((NPLANE * N,), jnp.float32),
    )(coarse_input.T, coarse_pred.T, fine_input.T, fine_pred.T)

    idxF = jnp.transpose(laplace_idx_list, (2, 0, 1)).reshape(-1)

    mesh = plsc.VectorSubcoreMesh(core_axis_name="c", subcore_axis_name="s")
    part = pl.kernel(
        _sc_body,
        mesh=mesh,
        compiler_params=pltpu.CompilerParams(needs_layout_passes=False),
        out_type=jax.ShapeDtypeStruct((NW * 16,), jnp.float32),
        scratch_types=[
            pltpu.VMEM((N,), jnp.float32),
            pltpu.VMEM(((KNB + 1) * CH,), jnp.int32),
            pltpu.VMEM(((KNB + 1) * CH,), jnp.int32),
            pltpu.VMEM((16,), jnp.float32),
            pltpu.SemaphoreType.DMA,
            pltpu.SemaphoreType.DMA,
        ],
    )(d_flat, idxF)
    return jnp.sum(part) * jnp.float32(0.5 / N)


# speedup vs baseline: 1.0057x; 1.0057x over previous
import jax
import jax.numpy as jnp
from jax import lax
from jax.experimental import pallas as pl
from jax.experimental.pallas import tpu as pltpu
from jax.experimental.pallas import tpu_sc as plsc

N = 50000
KNB = 8
NC = 2
NS = 16
NW = NC * NS
NPLANE = 6
NSUB = 5
SUB = N // NSUB
CH = 2000
NCHUNK = SUB // CH
CH32 = (CH // 32) * 32


def _diff_body(ci, cp, fi, fp, o_ref):
    for c in range(3):
        o_ref[pl.ds(c * N, N)] = ci[c, :] - cp[c, :]
        o_ref[pl.ds((3 + c) * N, N)] = fi[c, :] - fp[c, :]


def _fire_idx_chunk(idxF, ibuf, g, nstart, sem):
    return [
        pltpu.async_copy(
            idxF.at[pl.ds((k * 2 + g) * N + nstart, CH)],
            ibuf.at[pl.ds(slot * CH, CH)], sem)
        for slot, k in enumerate(list(range(KNB)) + [KNB + 1])
    ]


def _sc_body(d_flat, idxF, out_hbm, table, ibuf0, ibuf1, outv, semt, semi):
    ibufs = (ibuf0, ibuf1)
    wid = lax.axis_index("c") * NS + lax.axis_index("s")
    outv[...] = jnp.zeros((16,), jnp.float32)

    @pl.when(wid < NPLANE * NSUB)
    def _():
        p = wid % NPLANE
        sub = wid // NPLANE
        g = p // 3
        nbase = sub * SUB

        tcopy = pltpu.async_copy(d_flat.at[pl.ds(p * N, N)], table, semt)
        pend = _fire_idx_chunk(idxF, ibufs[0], g, nbase, semi)
        tcopy.wait()
        lossvec = jnp.zeros((16,), jnp.float32)
        for j in range(NCHUNK):
            nxt = None
            if j + 1 < NCHUNK:
                nxt = _fire_idx_chunk(
                    idxF, ibufs[(j + 1) % 2], g, nbase + (j + 1) * CH, semi)
            for h in pend:
                h.wait()
            ibuf = ibufs[j % 2]

            def sub16(o, lv, ibuf=ibuf, j=j):
                deg = ibuf[pl.ds(KNB * CH + o, 16)]
                inv = 1.0 / deg.astype(jnp.float32)
                acc = jnp.zeros((16,), jnp.float32)
                for k in range(KNB):
                    nb = ibuf[pl.ds(k * CH + o, 16)]
                    acc = acc + plsc.load_gather(table, [nb])
                own = table[pl.ds(nbase + j * CH + o, 16)]
                r = own - acc * inv
                return lv + r * r

            def body32(o, lv, sub16=sub16):
                return sub16(o + 16, sub16(o, lv))

            lossvec = plsc.parallel_loop(0, CH32, 32, carry=lossvec)(body32)
            for o in range(CH32, CH, 16):
                lossvec = sub16(o, lossvec)
            pend = nxt
        outv[...] = lossvec

    pltpu.sync_copy(outv, out_hbm.at[pl.ds(wid * 16, 16)])


def kernel(coarse_input, coarse_pred, fine_input, fine_pred, laplace_idx_list):
    d_flat = pl.pallas_call(
        _diff_body,
        out_shape=jax.ShapeDtypeStruct((NPLANE * N,), jnp.float32),
    )(coarse_input.T, coarse_pred.T, fine_input.T, fine_pred.T)

    idxF = jnp.transpose(laplace_idx_list, (2, 0, 1)).reshape(-1)

    mesh = plsc.VectorSubcoreMesh(core_axis_name="c", subcore_axis_name="s")
    part = pl.kernel(
        _sc_body,
        mesh=mesh,
        compiler_params=pltpu.CompilerParams(needs_layout_passes=False),
        out_type=jax.ShapeDtypeStruct((NW * 16,), jnp.float32),
        scratch_types=[
            pltpu.VMEM((N,), jnp.float32),
            pltpu.VMEM(((KNB + 1) * CH,), jnp.int32),
            pltpu.VMEM(((KNB + 1) * CH,), jnp.int32),
            pltpu.VMEM((16,), jnp.float32),
            pltpu.SemaphoreType.DMA,
            pltpu.SemaphoreType.DMA,
        ],
    )(d_flat, idxF)
    return jnp.sum(part) * jnp.float32(0.5 / N)
